# baseline (speedup 1.0000x reference)
import functools

import jax
import jax.numpy as jnp
from jax import lax
from jax.experimental import pallas as pl
from jax.experimental.pallas import tpu as pltpu
from jax.experimental.pallas import tpu_sc as plsc

N_NODES = 10000
N_HYPEREDGES = 5000
N_INC = 320000
D = 128

NC = 2
NS = 16
NW = NC * NS
L = 16
PER_W = 10240
CHUNK = 80
NPASS = 2
NCHP = PER_W // (NPASS * CHUNK)
HPAIR = NCHP // 2

HP = 5120
NP = 10240


def _make_sc_stage(n_tab, n_out, with_counts):
    rpt = n_out // NS
    mesh = plsc.VectorSubcoreMesh(
        core_axis_name="c", subcore_axis_name="s", num_cores=NC, num_subcores=NS
    )

    out_type = [jax.ShapeDtypeStruct((NC * n_out, D), jnp.float32)]
    scratch = [
        pltpu.VMEM((NCHP, CHUNK), jnp.int32),
        pltpu.VMEM((NCHP, CHUNK), jnp.int32),
        pltpu.VMEM((CHUNK, D), jnp.float32),
        pltpu.VMEM((CHUNK, D), jnp.float32),
        pltpu.VMEM_SHARED((n_out, D), jnp.float32),
        pltpu.SemaphoreType.DMA,
        pltpu.SemaphoreType.DMA,
    ]
    if with_counts:
        out_type += [
            jax.ShapeDtypeStruct((NW, n_out), jnp.float32),
            jax.ShapeDtypeStruct((NW, n_tab), jnp.float32),
        ]
        scratch += [
            pltpu.VMEM((n_out,), jnp.float32),
            pltpu.VMEM((n_tab,), jnp.float32),
        ]

    @functools.partial(
        pl.kernel,
        out_type=tuple(out_type) if with_counts else out_type[0],
        mesh=mesh,
        scratch_types=scratch,
        compiler_params=pltpu.CompilerParams(needs_layout_passes=False),
    )
    def sc_kernel(table, gidx4, sidx4, zrow, zcs, zcg, *rest):
        if with_counts:
            (out_sum, out_cs, out_cg,
             gv, sv, rows_a, rows_b, acc_sh, sem_a, sem_b, cs_v, cg_v) = rest
        else:
            out_sum, gv, sv, rows_a, rows_b, acc_sh, sem_a, sem_b = rest
        cid = lax.axis_index("c")
        sid = lax.axis_index("s")
        wid = sid * NC + cid

        if with_counts:
            pltpu.sync_copy(zcs, cs_v)
            pltpu.sync_copy(zcg, cg_v)
        pltpu.sync_copy(zrow, acc_sh.at[pl.ds(sid * rpt, rpt)])
        plsc.subcore_barrier()

        ones16 = jnp.ones((L,), jnp.float32)

        def counts(j):
            if with_counts:
                for k in range(CHUNK // L):
                    plsc.addupdate_scatter(
                        cs_v, [sv[j, pl.ds(k * L, L)]], ones16)
                    plsc.addupdate_scatter(
                        cg_v, [gv[j, pl.ds(k * L, L)]], ones16)

        def body(g, carry):
            j0 = 2 * g
            j1 = 2 * g + 1
            pltpu.make_async_copy(table.at[gv.at[j0]], rows_a, sem_a).wait()
            pltpu.async_copy(table.at[gv.at[j1]], rows_b, sem_b)
            pltpu.sync_copy(rows_a, acc_sh.at[sv.at[j0]], add=True)
            counts(j0)
            pltpu.make_async_copy(table.at[gv.at[j1]], rows_b, sem_b).wait()

            @pl.when(g + 1 < HPAIR)
            def _():
                pltpu.async_copy(table.at[gv.at[j0 + 2]], rows_a, sem_a)

            pltpu.sync_copy(rows_b, acc_sh.at[sv.at[j1]], add=True)
            counts(j1)
            return carry

        for p in range(NPASS):
            pltpu.sync_copy(gidx4.at[wid, p], gv)
            pltpu.sync_copy(sidx4.at[wid, p], sv)
            pltpu.async_copy(table.at[gv.at[0]], rows_a, sem_a)
            lax.fori_loop(0, HPAIR, body, 0)
        plsc.subcore_barrier()

        base = cid * n_out + sid * rpt
        pltpu.sync_copy(acc_sh.at[pl.ds(sid * rpt, rpt)],
                        out_sum.at[pl.ds(base, rpt)])
        if with_counts:
            pltpu.sync_copy(cs_v, out_cs.at[wid])
            pltpu.sync_copy(cg_v, out_cg.at[wid])

    return sc_kernel


_sc_stage1 = _make_sc_stage(NP, HP, True)
_sc_stage2 = _make_sc_stage(HP, NP, False)


def _tc1_body(s0, s1, c, w, b, o):
    cnt = jnp.sum(c[...], axis=0)[:, None]
    m = (s0[...] + s1[...]) / jnp.maximum(cnt, 1.0)
    y = jnp.dot(m, w[...], preferred_element_type=jnp.float32) + b[...]
    o[...] = jnp.maximum(y, 0.0)


def _tc2_body(s0, s1, c, nf, w, b, g, be, o):
    cnt = jnp.sum(c[...], axis=0)[:, None]
    m = (s0[...] + s1[...]) / jnp.maximum(cnt, 1.0)
    x = jnp.dot(m, w[...], preferred_element_type=jnp.float32) + b[...] + nf[...]
    mu = jnp.mean(x, axis=-1, keepdims=True)
    var = jnp.mean((x - mu) ** 2, axis=-1, keepdims=True)
    x = (x - mu) * lax.rsqrt(var + 1e-5) * g[...] + be[...]
    o[...] = jnp.maximum(x, 0.0)


def _tc_stage1(parts, cnts, wT, b):
    B = 640
    nb = HP // B
    return pl.pallas_call(
        _tc1_body,
        grid=(nb,),
        in_specs=[
            pl.BlockSpec((B, D), lambda i: (i, 0)),
            pl.BlockSpec((B, D), lambda i: (i + nb, 0)),
            pl.BlockSpec((NW, B), lambda i: (0, i)),
            pl.BlockSpec((D, D), lambda i: (0, 0)),
            pl.BlockSpec((1, D), lambda i: (0, 0)),
        ],
        out_specs=pl.BlockSpec((B, D), lambda i: (i, 0)),
        out_shape=jax.ShapeDtypeStruct((HP, D), jnp.float32),
    )(parts, parts, cnts, wT, b)


def _tc_stage2(parts, cnts, nf_pad, wT, b, g, be):
    B = 640
    nb = NP // B
    return pl.pallas_call(
        _tc2_body,
        grid=(nb,),
        in_specs=[
            pl.BlockSpec((B, D), lambda i: (i, 0)),
            pl.BlockSpec((B, D), lambda i: (i + nb, 0)),
            pl.BlockSpec((NW, B), lambda i: (0, i)),
            pl.BlockSpec((B, D), lambda i: (i, 0)),
            pl.BlockSpec((D, D), lambda i: (0, 0)),
            pl.BlockSpec((1, D), lambda i: (0, 0)),
            pl.BlockSpec((1, D), lambda i: (0, 0)),
            pl.BlockSpec((1, D), lambda i: (0, 0)),
        ],
        out_specs=pl.BlockSpec((B, D), lambda i: (i, 0)),
        out_shape=jax.ShapeDtypeStruct((NP, D), jnp.float32),
    )(parts, parts, cnts, nf_pad, wT, b, g, be)


def kernel(node_features, node_idx, hedge_idx, W_he, b_he, W_node, b_node,
           ln_gamma, ln_beta):
    pad = PER_W - N_INC // NW
    nidx = node_idx.astype(jnp.int32).reshape(NW, N_INC // NW)
    nidx = jnp.pad(nidx, ((0, 0), (0, pad)), constant_values=NP - 1)
    nidx = nidx.reshape(NW, NPASS, NCHP, CHUNK)
    hidx = hedge_idx.astype(jnp.int32).reshape(NW, N_INC // NW)
    hidx = jnp.pad(hidx, ((0, 0), (0, pad)), constant_values=HP - 1)
    hidx = hidx.reshape(NW, NPASS, NCHP, CHUNK)

    nf_pad = jnp.pad(node_features, ((0, NP - N_NODES), (0, 0)))

    z1r = jnp.zeros((HP // NS, D), jnp.float32)
    zcs = jnp.zeros((HP,), jnp.float32)
    zcg = jnp.zeros((NP,), jnp.float32)
    he_sum, he_cnt, nd_cnt = _sc_stage1(nf_pad, nidx, hidx, z1r, zcs, zcg)
    he_feat = _tc_stage1(he_sum, he_cnt, W_he.T, b_he.reshape(1, D))

    z2r = jnp.zeros((NP // NS, D), jnp.float32)
    nd_sum = _sc_stage2(he_feat, hidx, nidx, z2r, zcs, zcg)

    out = _tc_stage2(nd_sum, nd_cnt, nf_pad, W_node.T, b_node.reshape(1, D),
                     ln_gamma.reshape(1, D), ln_beta.reshape(1, D))
    return out[:N_NODES]
